# baseline (speedup 1.0000x reference)
import functools

import jax
import jax.numpy as jnp
from jax import lax
from jax.experimental import pallas as pl
from jax.experimental.pallas import tpu as pltpu
from jax.experimental.pallas import tpu_sc as plsc

NC = 2
NS = 16
NW = NC * NS

B, C, D3 = 4, 16, 64
Q = 16384
QTOT = B * Q
QPW = QTOT // NW
CH = 256
NCHUNK = QPW // CH
VOX = D3 * D3 * D3

_CORNER = (0, 1, 64, 65, 4096, 4097, 4160, 4161)
_SCALE = 105.0


def _body(tab_hbm, gx_hbm, gy_hbm, gz_hbm, out_hbm,
          gxv, gyv, gzv, wfx, wfy, wfz, idxv, rows, outb, sem):
    wid = lax.axis_index("s") * NC + lax.axis_index("c")
    qbase = wid * QPW
    boff = (qbase // Q) * VOX

    pltpu.sync_copy(gx_hbm.at[pl.ds(qbase, QPW)], gxv)
    pltpu.sync_copy(gy_hbm.at[pl.ds(qbase, QPW)], gyv)
    pltpu.sync_copy(gz_hbm.at[pl.ds(qbase, QPW)], gzv)

    def chunk(ch, carry):
        for i in range(CH // 16):
            off = ch * CH + 16 * i

            def axis_frac(gref, wref):
                ix = (gref[pl.ds(off, 16)] + 1.0) * 31.5
                i0 = jnp.clip(ix.astype(jnp.int32), 0, D3 - 2)
                wref[pl.ds(16 * i, 16)] = ix - i0.astype(jnp.float32)
                return i0

            ix0 = axis_frac(gxv, wfx)
            iy0 = axis_frac(gyv, wfy)
            iz0 = axis_frac(gzv, wfz)
            base = (iz0 * D3 + iy0) * D3 + ix0 + boff
            for k in range(8):
                f = k * CH + 16 * i
                idxv[f // 128, pl.ds(f % 128, 16)] = base + _CORNER[k]

        cps = [
            pltpu.async_copy(tab_hbm.at[idxv.at[r]],
                             rows.at[pl.ds(128 * r, 128)], sem)
            for r in range(8 * CH // 128)
        ]
        for cp in cps:
            cp.wait()

        def combine(t, c2):
            fxv = wfx[pl.ds(16 * t, 16)]
            fyv = wfy[pl.ds(16 * t, 16)]
            fzv = wfz[pl.ds(16 * t, 16)]
            for j in range(16):
                q = 16 * t + j
                fx = fxv[j]
                fy = fyv[j]
                fz = fzv[j]
                v = [rows[k * CH + q, :] for k in range(8)]
                d0 = v[1] - v[0]
                d1 = v[3] - v[2]
                d2 = v[5] - v[4]
                d3 = v[7] - v[6]
                c00 = v[0] + fx * d0
                c01 = v[2] + fx * d1
                c10 = v[4] + fx * d2
                c11 = v[6] + fx * d3
                e0 = c01 - c00
                e1 = c11 - c10
                c0 = c00 + fy * e0
                c1 = c10 + fy * e1
                ddz = c1 - c0
                u0 = d0 + fy * (d1 - d0)
                u1 = d2 + fy * (d3 - d2)
                outb[q, 0, :] = c0 + fz * ddz
                outb[q, 1, :] = (u0 + fz * (u1 - u0)) * _SCALE
                outb[q, 2, :] = (e0 + fz * (e1 - e0)) * _SCALE
                outb[q, 3, :] = ddz * _SCALE
            return c2

        lax.fori_loop(0, CH // 16, combine, 0)

        pltpu.sync_copy(outb, out_hbm.at[pl.ds(qbase + ch * CH, CH)])
        return carry

    lax.fori_loop(0, NCHUNK, chunk, 0)


def kernel(input, grid):
    tab = jnp.transpose(input, (0, 2, 3, 4, 1)).reshape(B * VOX, C)
    gx = grid[..., 0].reshape(QTOT)
    gy = grid[..., 1].reshape(QTOT)
    gz = grid[..., 2].reshape(QTOT)

    sck = pl.kernel(
        _body,
        out_type=jax.ShapeDtypeStruct((QTOT, 4, C), jnp.float32),
        mesh=plsc.VectorSubcoreMesh(
            core_axis_name="c", subcore_axis_name="s",
            num_cores=NC, num_subcores=NS),
        compiler_params=pltpu.CompilerParams(use_tc_tiling_on_sc=False),
        scratch_types=[
            pltpu.VMEM((QPW,), jnp.float32),
            pltpu.VMEM((QPW,), jnp.float32),
            pltpu.VMEM((QPW,), jnp.float32),
            pltpu.VMEM((CH,), jnp.float32),
            pltpu.VMEM((CH,), jnp.float32),
            pltpu.VMEM((CH,), jnp.float32),
            pltpu.VMEM((8 * CH // 128, 128), jnp.int32),
            pltpu.VMEM((8 * CH, C), jnp.float32),
            pltpu.VMEM((CH, 4, C), jnp.float32),
            pltpu.SemaphoreType.DMA,
        ],
    )
    out = sck(tab, gx, gy, gz)
    return jnp.transpose(out.reshape(B, Q, 4, C), (0, 3, 2, 1))


# speedup vs baseline: 1.1122x; 1.1122x over previous
import jax
import jax.numpy as jnp
from jax import lax
from jax.experimental import pallas as pl
from jax.experimental.pallas import tpu as pltpu
from jax.experimental.pallas import tpu_sc as plsc

NC = 2
NS = 16
NW = NC * NS

B, C, D3 = 4, 16, 64
Q = 16384
QTOT = B * Q
QPW = QTOT // NW
CH = 256
NCHUNK = QPW // CH
VOX = D3 * D3 * D3
VPW = B * VOX // NW
VR = 1024
NTCH = VPW // VR

_CORNER = (0, 1, 64, 65, 4096, 4097, 4160, 4161)
_SCALE = 105.0


def _body(vol_hbm, grid_hbm, out_hbm, tab_hbm,
          gbuf, tin, wfx, wfy, wfz, idxv, rows, outb, sem, sem2):
    ci = lax.axis_index("c")
    si = lax.axis_index("s")
    qbase = ci * (2 * Q) + si * QPW
    vbase = ci * (2 * VOX) + si * VPW

    gcp = pltpu.async_copy(grid_hbm.at[pl.ds(qbase * 3, 3 * QPW)], gbuf, sem2)

    lane = lax.iota(jnp.int32, 16)

    def tchunk(t, _):
        v0 = vbase + t * VR
        bt = v0 // VOX
        vloc = v0 - bt * VOX
        cps = [
            pltpu.async_copy(
                vol_hbm.at[pl.ds((bt * C + c) * VOX + vloc, VR)],
                tin.at[c], sem)
            for c in range(C)
        ]
        for cp in cps:
            cp.wait()

        def jgroup(j, _2):
            rr = 16 * j + lane
            for c in range(C):
                vv = tin[c, pl.ds(16 * j, 16)]
                plsc.store_scatter(rows, [rr, jnp.full((16,), c, jnp.int32)],
                                   vv)
            return _2

        lax.fori_loop(0, VR // 16, jgroup, 0)
        pltpu.sync_copy(rows.at[pl.ds(0, VR)], tab_hbm.at[pl.ds(v0, VR)])
        return _

    lax.fori_loop(0, NTCH, tchunk, 0)
    gcp.wait()
    plsc.subcore_barrier()

    boff = (qbase // Q) * VOX
    lane3 = lane * 3

    def chunk(ch, carry):
        for i in range(CH // 16):
            off3 = (ch * CH + 16 * i) * 3

            def axis_frac(comp, wref):
                g = plsc.load_gather(gbuf, [lane3 + (off3 + comp)])
                ix = (g + 1.0) * 31.5
                i0 = jnp.clip(ix.astype(jnp.int32), 0, D3 - 2)
                wref[pl.ds(16 * i, 16)] = ix - i0.astype(jnp.float32)
                return i0

            ix0 = axis_frac(0, wfx)
            iy0 = axis_frac(1, wfy)
            iz0 = axis_frac(2, wfz)
            base = (iz0 * D3 + iy0) * D3 + ix0 + boff
            for k in range(8):
                f = k * CH + 16 * i
                idxv[f // 128, pl.ds(f % 128, 16)] = base + _CORNER[k]

        cps = [
            pltpu.async_copy(tab_hbm.at[idxv.at[r]],
                             rows.at[pl.ds(128 * r, 128)], sem)
            for r in range(8 * CH // 128)
        ]
        for cp in cps:
            cp.wait()

        def combine(t, c2):
            fxv = wfx[pl.ds(16 * t, 16)]
            fyv = wfy[pl.ds(16 * t, 16)]
            fzv = wfz[pl.ds(16 * t, 16)]
            for j in range(16):
                q = 16 * t + j
                fx = fxv[j]
                fy = fyv[j]
                fz = fzv[j]
                v = [rows[k * CH + q, :] for k in range(8)]
                d0 = v[1] - v[0]
                d1 = v[3] - v[2]
                d2 = v[5] - v[4]
                d3 = v[7] - v[6]
                c00 = v[0] + fx * d0
                c01 = v[2] + fx * d1
                c10 = v[4] + fx * d2
                c11 = v[6] + fx * d3
                e0 = c01 - c00
                e1 = c11 - c10
                c0 = c00 + fy * e0
                c1 = c10 + fy * e1
                ddz = c1 - c0
                u0 = d0 + fy * (d1 - d0)
                u1 = d2 + fy * (d3 - d2)
                outb[q, 0, :] = c0 + fz * ddz
                outb[q, 1, :] = (u0 + fz * (u1 - u0)) * _SCALE
                outb[q, 2, :] = (e0 + fz * (e1 - e0)) * _SCALE
                outb[q, 3, :] = ddz * _SCALE
            return c2

        lax.fori_loop(0, CH // 16, combine, 0)

        pltpu.sync_copy(outb, out_hbm.at[pl.ds(qbase + ch * CH, CH)])
        return carry

    lax.fori_loop(0, NCHUNK, chunk, 0)


def kernel(input, grid):
    vol = input.reshape(B * C * VOX)
    gflat = grid.reshape(QTOT * 3)

    sck = pl.kernel(
        _body,
        out_type=(
            jax.ShapeDtypeStruct((QTOT, 4, C), jnp.float32),
            jax.ShapeDtypeStruct((B * VOX, C), jnp.float32),
        ),
        mesh=plsc.VectorSubcoreMesh(
            core_axis_name="c", subcore_axis_name="s",
            num_cores=NC, num_subcores=NS),
        compiler_params=pltpu.CompilerParams(
            use_tc_tiling_on_sc=False, needs_layout_passes=False),
        scratch_types=[
            pltpu.VMEM((3 * QPW,), jnp.float32),
            pltpu.VMEM((C, VR), jnp.float32),
            pltpu.VMEM((CH,), jnp.float32),
            pltpu.VMEM((CH,), jnp.float32),
            pltpu.VMEM((CH,), jnp.float32),
            pltpu.VMEM((8 * CH // 128, 128), jnp.int32),
            pltpu.VMEM((8 * CH, C), jnp.float32),
            pltpu.VMEM((CH, 4, C), jnp.float32),
            pltpu.SemaphoreType.DMA,
            pltpu.SemaphoreType.DMA,
        ],
    )
    out, _ = sck(vol, gflat)
    return jnp.transpose(out.reshape(B, Q, 4, C), (0, 3, 2, 1))


# speedup vs baseline: 1.5796x; 1.4202x over previous
import jax
import jax.numpy as jnp
from jax import lax
from jax.experimental import pallas as pl
from jax.experimental.pallas import tpu as pltpu
from jax.experimental.pallas import tpu_sc as plsc

NC = 2
NS = 16
NW = NC * NS

B, C, D3 = 4, 16, 64
Q = 16384
QTOT = B * Q
QPW = QTOT // NW
CH = 128
NCHQ = QPW // CH
VOX = D3 * D3 * D3
VPW = B * VOX // NW
HP = 1024
NTCH = VPW // HP

_CORNER = (0, 1, 64, 65, 4096, 4097, 4160, 4161)
_SCALE = 105.0


def _body(vol_hbm, g_hbm, out_hbm, tab_hbm,
          gbuf, tin, tstage, wfx, wfy, wfz, idxv, rows, outb,
          semg, semia, semib, semwa, semwb, semga, semgb, semda, semdb):
    ci = lax.axis_index("c")
    si = lax.axis_index("s")
    qbase = ci * (2 * Q) + si * QPW
    vbase = ci * (2 * VOX) + si * VPW

    gcps = [
        pltpu.async_copy(g_hbm.at[pl.ds(comp * QTOT + qbase, QPW)],
                         gbuf.at[comp], semg)
        for comp in range(3)
    ]

    lane = lax.iota(jnp.int32, 16)

    def t_src(t, c):
        v0 = vbase + t * HP
        bt = v0 // VOX
        vloc = v0 - bt * VOX
        z = vloc // (D3 * D3)
        yq = (vloc - z * D3 * D3) // HP
        return vol_hbm.at[bt, c, z, pl.ds(16 * yq, 16), :]

    def t_fire(t, p, sem):
        for c in range(C):
            pltpu.async_copy(t_src(t, c), tin.at[p, c], sem)

    def t_wait(p, sem):
        for c in range(C):
            pltpu.make_async_copy(vol_hbm.at[0, 0, 0, pl.ds(0, 16), :],
                                  tin.at[p, c], sem).wait()

    def t_scatter(p):
        pv = jnp.full((16,), p, jnp.int32)

        def jgroup(j, _2):
            rr = 16 * j + lane
            yy = j // 4
            xo = 16 * (j % 4)
            for c in range(C):
                vv = tin[p, c, yy, pl.ds(xo, 16)]
                plsc.store_scatter(
                    tstage, [pv, rr, jnp.full((16,), c, jnp.int32)], vv)
            return _2

        lax.fori_loop(0, HP // 16, jgroup, 0)

    def t_write(t, p, sem):
        pltpu.async_copy(tstage.at[p], tab_hbm.at[pl.ds(vbase + t * HP, HP)],
                         sem)

    def t_wdrain(sem):
        pltpu.make_async_copy(tstage.at[0], tab_hbm.at[pl.ds(0, HP)],
                              sem).wait()

    t_fire(0, 0, semia)

    def t_loop(t2, carry):
        c0 = 2 * t2
        t_fire(c0 + 1, 1, semib)
        t_wait(0, semia)

        @pl.when(t2 > 0)
        def _():
            t_wdrain(semwa)

        t_scatter(0)
        t_write(c0, 0, semwa)

        @pl.when(t2 < NTCH // 2 - 1)
        def _():
            t_fire(c0 + 2, 0, semia)

        t_wait(1, semib)

        @pl.when(t2 > 0)
        def _():
            t_wdrain(semwb)

        t_scatter(1)
        t_write(c0 + 1, 1, semwb)
        return carry

    lax.fori_loop(0, NTCH // 2, t_loop, 0)
    t_wdrain(semwa)
    t_wdrain(semwb)
    for cp in gcps:
        cp.wait()
    plsc.subcore_barrier()

    bq = qbase // Q
    boff = bq * VOX
    qoff = qbase - bq * Q

    def q_fire(ch, p, sem):
        for i in range(CH // 16):
            off = ch * CH + 16 * i

            def axis_frac(comp, wref):
                g = gbuf[comp, pl.ds(off, 16)]
                ix = (g + 1.0) * 31.5
                i0 = jnp.clip(ix.astype(jnp.int32), 0, D3 - 2)
                wref[p, pl.ds(16 * i, 16)] = ix - i0.astype(jnp.float32)
                return i0

            ix0 = axis_frac(0, wfx)
            iy0 = axis_frac(1, wfy)
            iz0 = axis_frac(2, wfz)
            base = (iz0 * D3 + iy0) * D3 + ix0 + boff
            for k in range(8):
                idxv[p, k, pl.ds(16 * i, 16)] = base + _CORNER[k]
        for r in range(8):
            pltpu.async_copy(tab_hbm.at[idxv.at[p, r]],
                             rows.at[p, pl.ds(128 * r, 128)], sem)

    def q_gwait(p, sem):
        for r in range(8):
            pltpu.make_async_copy(tab_hbm.at[idxv.at[p, r]],
                                  rows.at[p, pl.ds(128 * r, 128)],
                                  sem).wait()

    def q_combine(p):
        pv = jnp.full((16,), p, jnp.int32)

        def combine(t, c2):
            fxv = wfx[p, pl.ds(16 * t, 16)]
            fyv = wfy[p, pl.ds(16 * t, 16)]
            fzv = wfz[p, pl.ds(16 * t, 16)]
            for j in range(16):
                q = 16 * t + j
                fx = fxv[j]
                fy = fyv[j]
                fz = fzv[j]
                v = [rows[p, k * CH + q, :] for k in range(8)]
                d0 = v[1] - v[0]
                d1 = v[3] - v[2]
                d2 = v[5] - v[4]
                d3 = v[7] - v[6]
                c00 = v[0] + fx * d0
                c01 = v[2] + fx * d1
                c10 = v[4] + fx * d2
                c11 = v[6] + fx * d3
                e0 = c01 - c00
                e1 = c11 - c10
                c0 = c00 + fy * e0
                c1 = c10 + fy * e1
                ddz = c1 - c0
                u0 = d0 + fy * (d1 - d0)
                u1 = d2 + fy * (d3 - d2)
                qv = jnp.full((16,), q, jnp.int32)
                vals = (c0 + fz * ddz,
                        (u0 + fz * (u1 - u0)) * _SCALE,
                        (e0 + fz * (e1 - e0)) * _SCALE,
                        ddz * _SCALE)
                for jo in range(4):
                    plsc.store_scatter(
                        outb,
                        [pv, lane, jnp.full((16,), jo, jnp.int32), qv],
                        vals[jo])
            return c2

        lax.fori_loop(0, CH // 16, combine, 0)

    def q_write(ch, p, sem):
        pltpu.async_copy(outb.at[p],
                         out_hbm.at[bq, :, :, pl.ds(qoff + ch * CH, CH)],
                         sem)

    def q_wdrain(sem):
        pltpu.make_async_copy(outb.at[0],
                              out_hbm.at[bq, :, :, pl.ds(qoff, CH)],
                              sem).wait()

    q_fire(0, 0, semga)

    def q_loop(t2, carry):
        c0 = 2 * t2
        q_fire(c0 + 1, 1, semgb)
        q_gwait(0, semga)

        @pl.when(t2 > 0)
        def _():
            q_wdrain(semda)

        q_combine(0)
        q_write(c0, 0, semda)

        @pl.when(t2 < NCHQ // 2 - 1)
        def _():
            q_fire(c0 + 2, 0, semga)

        q_gwait(1, semgb)

        @pl.when(t2 > 0)
        def _():
            q_wdrain(semdb)

        q_combine(1)
        q_write(c0 + 1, 1, semdb)
        return carry

    lax.fori_loop(0, NCHQ // 2, q_loop, 0)
    q_wdrain(semda)
    q_wdrain(semdb)


def kernel(input, grid):
    gxyz = jnp.transpose(grid, (2, 0, 1)).reshape(3 * QTOT)

    sck = pl.kernel(
        _body,
        out_type=(
            jax.ShapeDtypeStruct((B, C, 4, Q), jnp.float32),
            jax.ShapeDtypeStruct((B * VOX, C), jnp.float32),
        ),
        mesh=plsc.VectorSubcoreMesh(
            core_axis_name="c", subcore_axis_name="s",
            num_cores=NC, num_subcores=NS),
        compiler_params=pltpu.CompilerParams(
            use_tc_tiling_on_sc=False, needs_layout_passes=False),
        scratch_types=[
            pltpu.VMEM((3, QPW), jnp.float32),
            pltpu.VMEM((2, C, 16, D3), jnp.float32),
            pltpu.VMEM((2, HP, C), jnp.float32),
            pltpu.VMEM((2, CH), jnp.float32),
            pltpu.VMEM((2, CH), jnp.float32),
            pltpu.VMEM((2, CH), jnp.float32),
            pltpu.VMEM((2, 8, CH), jnp.int32),
            pltpu.VMEM((2, 8 * CH, C), jnp.float32),
            pltpu.VMEM((2, C, 4, CH), jnp.float32),
            pltpu.SemaphoreType.DMA,
            pltpu.SemaphoreType.DMA,
            pltpu.SemaphoreType.DMA,
            pltpu.SemaphoreType.DMA,
            pltpu.SemaphoreType.DMA,
            pltpu.SemaphoreType.DMA,
            pltpu.SemaphoreType.DMA,
            pltpu.SemaphoreType.DMA,
            pltpu.SemaphoreType.DMA,
        ],
    )
    out, _ = sck(input, gxyz)
    return out


# speedup vs baseline: 2.0234x; 1.2810x over previous
import jax
import jax.numpy as jnp
from jax import lax
from jax.experimental import pallas as pl
from jax.experimental.pallas import tpu as pltpu
from jax.experimental.pallas import tpu_sc as plsc

NC = 2
NS = 16
NW = NC * NS

B, C, D3 = 4, 16, 64
Q = 16384
QTOT = B * Q
QPW = QTOT // NW
CH = 128
NCHQ = QPW // CH
VOX = D3 * D3 * D3
VPW = B * VOX // NW
HP = 1024
NTCH = VPW // HP

_CORNER = (0, 1, 64, 65, 4096, 4097, 4160, 4161)
_SCALE = 105.0


def _body(vol_hbm, g_hbm, out_hbm, tab_hbm,
          gbuf, tin, tstage, wfx, wfy, wfz, idxv, rows, outb,
          semg, semia, semib, semwa, semwb, semga, semgb, semda, semdb):
    ci = lax.axis_index("c")
    si = lax.axis_index("s")
    qbase = ci * (2 * Q) + si * QPW
    vbase = ci * (2 * VOX) + si * VPW

    gcps = [
        pltpu.async_copy(g_hbm.at[pl.ds(comp * QTOT + qbase, QPW)],
                         gbuf.at[comp], semg)
        for comp in range(3)
    ]

    lane = lax.iota(jnp.int32, 16)

    def t_src(t, c):
        v0 = vbase + t * HP
        bt = v0 // VOX
        vloc = v0 - bt * VOX
        z = vloc // (D3 * D3)
        yq = (vloc - z * D3 * D3) // HP
        return vol_hbm.at[bt, c, z, pl.ds(16 * yq, 16), :]

    def t_fire(t, p, sem):
        for c in range(C):
            pltpu.async_copy(t_src(t, c), tin.at[p, c], sem)

    def t_wait(p, sem):
        for c in range(C):
            pltpu.make_async_copy(vol_hbm.at[0, 0, 0, pl.ds(0, 16), :],
                                  tin.at[p, c], sem).wait()

    cvs = [jnp.full((16,), c, jnp.int32) for c in range(C)]

    def t_scatter(p):
        pv = jnp.full((16,), p, jnp.int32)

        @plsc.parallel_loop(0, HP // 16)
        def jgroup(j):
            rr = 16 * j + lane
            yy = j // 4
            xo = 16 * (j % 4)
            for c in range(C):
                vv = tin[p, c, yy, pl.ds(xo, 16)]
                plsc.store_scatter(tstage, [pv, rr, cvs[c]], vv)

    def t_write(t, p, sem):
        pltpu.async_copy(tstage.at[p], tab_hbm.at[pl.ds(vbase + t * HP, HP)],
                         sem)

    def t_wdrain(sem):
        pltpu.make_async_copy(tstage.at[0], tab_hbm.at[pl.ds(0, HP)],
                              sem).wait()

    t_fire(0, 0, semia)

    def t_loop(t2, carry):
        c0 = 2 * t2
        t_fire(c0 + 1, 1, semib)
        t_wait(0, semia)

        @pl.when(t2 > 0)
        def _():
            t_wdrain(semwa)

        t_scatter(0)
        t_write(c0, 0, semwa)

        @pl.when(t2 < NTCH // 2 - 1)
        def _():
            t_fire(c0 + 2, 0, semia)

        t_wait(1, semib)

        @pl.when(t2 > 0)
        def _():
            t_wdrain(semwb)

        t_scatter(1)
        t_write(c0 + 1, 1, semwb)
        return carry

    lax.fori_loop(0, NTCH // 2, t_loop, 0)
    t_wdrain(semwa)
    t_wdrain(semwb)
    for cp in gcps:
        cp.wait()
    plsc.subcore_barrier()

    bq = qbase // Q
    boff = bq * VOX
    qoff = qbase - bq * Q

    def q_fire(ch, p, sem):
        for i in range(CH // 16):
            off = ch * CH + 16 * i

            def axis_frac(comp, wref):
                g = gbuf[comp, pl.ds(off, 16)]
                ix = (g + 1.0) * 31.5
                i0 = jnp.clip(ix.astype(jnp.int32), 0, D3 - 2)
                wref[p, pl.ds(16 * i, 16)] = ix - i0.astype(jnp.float32)
                return i0

            ix0 = axis_frac(0, wfx)
            iy0 = axis_frac(1, wfy)
            iz0 = axis_frac(2, wfz)
            base = (iz0 * D3 + iy0) * D3 + ix0 + boff
            for k in range(8):
                idxv[p, k, pl.ds(16 * i, 16)] = base + _CORNER[k]
        for r in range(8):
            pltpu.async_copy(tab_hbm.at[idxv.at[p, r]],
                             rows.at[p, pl.ds(128 * r, 128)], sem)

    def q_gwait(p, sem):
        for r in range(8):
            pltpu.make_async_copy(tab_hbm.at[idxv.at[p, r]],
                                  rows.at[p, pl.ds(128 * r, 128)],
                                  sem).wait()

    jovs = [jnp.full((16,), jo, jnp.int32) for jo in range(4)]

    def q_combine(p):
        pv = jnp.full((16,), p, jnp.int32)

        @plsc.parallel_loop(0, CH // 16)
        def combine(t):
            fxv = wfx[p, pl.ds(16 * t, 16)]
            fyv = wfy[p, pl.ds(16 * t, 16)]
            fzv = wfz[p, pl.ds(16 * t, 16)]
            for j in range(16):
                q = 16 * t + j
                fx = fxv[j]
                fy = fyv[j]
                fz = fzv[j]
                v = [rows[p, k * CH + q, :] for k in range(8)]
                d0 = v[1] - v[0]
                d1 = v[3] - v[2]
                d2 = v[5] - v[4]
                d3 = v[7] - v[6]
                c00 = v[0] + fx * d0
                c01 = v[2] + fx * d1
                c10 = v[4] + fx * d2
                c11 = v[6] + fx * d3
                e0 = c01 - c00
                e1 = c11 - c10
                c0 = c00 + fy * e0
                c1 = c10 + fy * e1
                ddz = c1 - c0
                u0 = d0 + fy * (d1 - d0)
                u1 = d2 + fy * (d3 - d2)
                qv = jnp.full((16,), q, jnp.int32)
                vals = (c0 + fz * ddz,
                        (u0 + fz * (u1 - u0)) * _SCALE,
                        (e0 + fz * (e1 - e0)) * _SCALE,
                        ddz * _SCALE)
                for jo in range(4):
                    plsc.store_scatter(outb, [pv, lane, jovs[jo], qv],
                                       vals[jo])

    def q_write(ch, p, sem):
        pltpu.async_copy(outb.at[p],
                         out_hbm.at[bq, :, :, pl.ds(qoff + ch * CH, CH)],
                         sem)

    def q_wdrain(sem):
        pltpu.make_async_copy(outb.at[0],
                              out_hbm.at[bq, :, :, pl.ds(qoff, CH)],
                              sem).wait()

    q_fire(0, 0, semga)

    def q_loop(t2, carry):
        c0 = 2 * t2
        q_fire(c0 + 1, 1, semgb)
        q_gwait(0, semga)

        @pl.when(t2 > 0)
        def _():
            q_wdrain(semda)

        q_combine(0)
        q_write(c0, 0, semda)

        @pl.when(t2 < NCHQ // 2 - 1)
        def _():
            q_fire(c0 + 2, 0, semga)

        q_gwait(1, semgb)

        @pl.when(t2 > 0)
        def _():
            q_wdrain(semdb)

        q_combine(1)
        q_write(c0 + 1, 1, semdb)
        return carry

    lax.fori_loop(0, NCHQ // 2, q_loop, 0)
    q_wdrain(semda)
    q_wdrain(semdb)


def kernel(input, grid):
    gxyz = jnp.transpose(grid, (2, 0, 1)).reshape(3 * QTOT)

    sck = pl.kernel(
        _body,
        out_type=(
            jax.ShapeDtypeStruct((B, C, 4, Q), jnp.float32),
            jax.ShapeDtypeStruct((B * VOX, C), jnp.float32),
        ),
        mesh=plsc.VectorSubcoreMesh(
            core_axis_name="c", subcore_axis_name="s",
            num_cores=NC, num_subcores=NS),
        compiler_params=pltpu.CompilerParams(
            use_tc_tiling_on_sc=False, needs_layout_passes=False),
        scratch_types=[
            pltpu.VMEM((3, QPW), jnp.float32),
            pltpu.VMEM((2, C, 16, D3), jnp.float32),
            pltpu.VMEM((2, HP, C), jnp.float32),
            pltpu.VMEM((2, CH), jnp.float32),
            pltpu.VMEM((2, CH), jnp.float32),
            pltpu.VMEM((2, CH), jnp.float32),
            pltpu.VMEM((2, 8, CH), jnp.int32),
            pltpu.VMEM((2, 8 * CH, C), jnp.float32),
            pltpu.VMEM((2, C, 4, CH), jnp.float32),
            pltpu.SemaphoreType.DMA,
            pltpu.SemaphoreType.DMA,
            pltpu.SemaphoreType.DMA,
            pltpu.SemaphoreType.DMA,
            pltpu.SemaphoreType.DMA,
            pltpu.SemaphoreType.DMA,
            pltpu.SemaphoreType.DMA,
            pltpu.SemaphoreType.DMA,
            pltpu.SemaphoreType.DMA,
        ],
    )
    out, _ = sck(input, gxyz)
    return out
